# baseline (speedup 1.0000x reference)
import functools

import jax
import jax.numpy as jnp
from jax import lax
from jax.experimental import pallas as pl
from jax.experimental.pallas import tpu as pltpu
from jax.experimental.pallas import tpu_sc as plsc

N = 10000
FEAT = 128
HID = 256
EDGES = 320000
KHOPS = 10
ALPHA = 0.1

NPAD = 10240
NCORES = 2
NSUB = 16
NTILES = NCORES * NSUB
EPT = 10496
EPAD = EPT * NTILES
CHUNK = 128
NCHUNKS = EPT // CHUNK
CGRID = 8
CROWS = NPAD // CGRID
RPT = NPAD // NSUB
GRID = NPAD // 128


def _deg_body(dst_hbm, out_hbm, didx_b, ones_v, zbuf_v, deg_sh,
              dsem0, dsem1):
    c = lax.axis_index("c")
    s = lax.axis_index("s")
    base = (c * NSUB + s) * EPT
    dsems = (dsem0, dsem1)
    for k in range(RPT // 16):
        zbuf_v[pl.ds(k * 16, 16)] = jnp.zeros((16,), jnp.float32)
    for k in range(CHUNK // 16):
        ones_v[pl.ds(k * 16, 16)] = jnp.ones((16,), jnp.float32)
    pltpu.sync_copy(zbuf_v, deg_sh.at[pl.ds(s * RPT, RPT)])
    plsc.subcore_barrier()

    def dfire(j, b):
        pltpu.async_copy(dst_hbm.at[pl.ds(base + j * CHUNK, CHUNK)],
                         didx_b.at[b], dsems[b])

    def dwait(b):
        pltpu.make_async_copy(dst_hbm.at[pl.ds(base, CHUNK)],
                              didx_b.at[b], dsems[b]).wait()

    dfire(0, 0)
    dfire(1, 1)

    def body(i, carry):
        j = 2 * i
        dwait(0)
        pltpu.sync_copy(ones_v, deg_sh.at[didx_b.at[0]], add=True)
        dfire(j + 2, 0)
        dwait(1)
        pltpu.sync_copy(ones_v, deg_sh.at[didx_b.at[1]], add=True)
        dfire(j + 3, 1)
        return carry

    lax.fori_loop(0, NCHUNKS // 2 - 1, body, 0)
    dwait(0)
    pltpu.sync_copy(ones_v, deg_sh.at[didx_b.at[0]], add=True)
    dwait(1)
    pltpu.sync_copy(ones_v, deg_sh.at[didx_b.at[1]], add=True)
    plsc.subcore_barrier()
    pltpu.sync_copy(deg_sh.at[pl.ds(s * RPT, RPT)],
                    out_hbm.at[c, pl.ds(s * RPT, RPT)])


def _scat_body(p_hbm, src_hbm, dst_hbm, out_hbm,
               sidx_all, didx_b, rows_v, s_sh, gsem0, gsem1, dsem0, dsem1,
               ssem0, ssem1):
    c = lax.axis_index("c")
    s = lax.axis_index("s")
    base = (c * NSUB + s) * EPT
    pltpu.sync_copy(src_hbm.at[pl.ds(base, EPT)], sidx_all)
    gsems = (gsem0, gsem1)
    dsems = (dsem0, dsem1)
    ssems = (ssem0, ssem1)

    def zb(i, carry):
        for k in range(8):
            rows_v[0, i, pl.ds(k * 16, 16)] = jnp.zeros((16,), jnp.float32)
        return carry

    lax.fori_loop(0, 128, zb, 0)
    for t in range(RPT // 128):
        pltpu.sync_copy(rows_v.at[0], s_sh.at[pl.ds(s * RPT + t * 128, 128)])
    plsc.subcore_barrier()

    def dfire(j, b):
        pltpu.async_copy(dst_hbm.at[pl.ds(base + j * CHUNK, CHUNK)],
                         didx_b.at[b], dsems[b])

    def dwait(b):
        pltpu.make_async_copy(dst_hbm.at[pl.ds(base, CHUNK)],
                              didx_b.at[b], dsems[b]).wait()

    def gfire(j, b):
        pltpu.async_copy(p_hbm.at[sidx_all.at[pl.ds(j * CHUNK, CHUNK)]],
                         rows_v.at[b], gsems[b])

    def gwait(b):
        pltpu.make_async_copy(p_hbm.at[sidx_all.at[pl.ds(0, CHUNK)]],
                              rows_v.at[b], gsems[b]).wait()

    def sfire(b):
        pltpu.async_copy(rows_v.at[b], s_sh.at[didx_b.at[b]], ssems[b],
                         add=True)

    def swait(b):
        pltpu.make_async_copy(rows_v.at[b], s_sh.at[didx_b.at[b]],
                              ssems[b]).wait()

    dfire(0, 0)
    dfire(1, 1)
    gfire(0, 0)
    gfire(1, 1)

    def body(i, carry):
        j = 2 * i
        gwait(0)
        dwait(0)
        sfire(0)
        gwait(1)
        dwait(1)
        sfire(1)
        swait(0)
        gfire(j + 2, 0)
        dfire(j + 2, 0)
        swait(1)
        gfire(j + 3, 1)
        dfire(j + 3, 1)
        return carry

    lax.fori_loop(0, NCHUNKS // 2 - 1, body, 0)
    gwait(0)
    dwait(0)
    sfire(0)
    gwait(1)
    dwait(1)
    sfire(1)
    swait(0)
    swait(1)
    plsc.subcore_barrier()
    pltpu.sync_copy(s_sh.at[pl.ds(s * RPT, RPT)],
                    out_hbm.at[c, pl.ds(s * RPT, RPT)])


def _mlp_body(xb, w1b, b1b, w2b, b2b, dpb, p0b, cbb):
    h = jnp.maximum(
        jnp.dot(xb[...], w1b[...], preferred_element_type=jnp.float32)
        + b1b[...], 0.0)
    h0 = jnp.dot(h, w2b[...], preferred_element_type=jnp.float32) + b2b[...]
    dp = dpb[...]
    deg = dp[0:1, :] + dp[1:2, :]
    dis = lax.rsqrt(deg)
    ir = lax.broadcasted_iota(jnp.int32, (128, 128), 0)
    ic = lax.broadcasted_iota(jnp.int32, (128, 128), 1)
    dmat = jnp.where(ir == ic, dis, 0.0)
    p0 = jnp.dot(dmat, h0, preferred_element_type=jnp.float32)
    dcol = jnp.dot(dmat, jnp.ones((128, 128), jnp.float32),
                   preferred_element_type=jnp.float32)
    p0b[...] = p0
    cbb[...] = 0.9 * dcol * dcol


def _comb_body(sb, p0b, cbb, ob):
    sblk = sb[...]
    ob[...] = cbb[...] * (sblk[0] + sblk[1]) + ALPHA * p0b[...]


def _fin_body(sb, p0b, cbb, ob):
    sblk = sb[...]
    cb = cbb[...]
    plast = cb * (sblk[0] + sblk[1]) + ALPHA * p0b[...]
    h = plast * lax.rsqrt(cb * (1.0 / 0.9))
    m = jnp.max(h, axis=1, keepdims=True)
    sh = h - m
    lse = jnp.log(jnp.sum(jnp.exp(sh), axis=1, keepdims=True))
    ob[...] = sh - lse


def kernel(x, edge_index, W1, b1, W2, b2):
    src = edge_index[0]
    dst = edge_index[1]
    loop = jnp.arange(NPAD, dtype=jnp.int32)
    npe = EPAD - EDGES - NPAD
    ar = jnp.arange(npe, dtype=jnp.int32)
    srcp = jnp.concatenate([src, loop, ar % N])
    dstp = jnp.concatenate([dst, loop, N + ar % (NPAD - N)])
    xp = jnp.pad(x, ((0, NPAD - N), (0, 0)))
    b1r = b1.reshape(1, HID)
    b2r = b2.reshape(1, FEAT)

    mesh = plsc.VectorSubcoreMesh(core_axis_name="c", subcore_axis_name="s")

    deg_call = pl.kernel(
        _deg_body,
        out_type=jax.ShapeDtypeStruct((NCORES, NPAD), jnp.float32),
        mesh=mesh,
        scratch_types=[
            pltpu.VMEM((2, CHUNK), jnp.int32),
            pltpu.VMEM((CHUNK,), jnp.float32),
            pltpu.VMEM((RPT,), jnp.float32),
            pltpu.VMEM_SHARED((NPAD,), jnp.float32),
            pltpu.SemaphoreType.DMA,
            pltpu.SemaphoreType.DMA,
        ],
    )
    degp = deg_call(dstp)

    mlp = pl.pallas_call(
        _mlp_body,
        grid=(GRID,),
        in_specs=[
            pl.BlockSpec((128, FEAT), lambda i: (i, 0)),
            pl.BlockSpec((FEAT, HID), lambda i: (0, 0)),
            pl.BlockSpec((1, HID), lambda i: (0, 0)),
            pl.BlockSpec((HID, FEAT), lambda i: (0, 0)),
            pl.BlockSpec((1, FEAT), lambda i: (0, 0)),
            pl.BlockSpec((NCORES, 128), lambda i: (0, i)),
        ],
        out_specs=[
            pl.BlockSpec((128, FEAT), lambda i: (i, 0)),
            pl.BlockSpec((128, FEAT), lambda i: (i, 0)),
        ],
        out_shape=[
            jax.ShapeDtypeStruct((NPAD, FEAT), jnp.float32),
            jax.ShapeDtypeStruct((NPAD, FEAT), jnp.float32),
        ],
    )
    p0, cB = mlp(xp, W1, b1r, W2, b2r, degp)

    scat = pl.kernel(
        _scat_body,
        out_type=jax.ShapeDtypeStruct((NCORES, NPAD, FEAT), jnp.float32),
        mesh=mesh,
        scratch_types=[
            pltpu.VMEM((EPT,), jnp.int32),
            pltpu.VMEM((2, CHUNK), jnp.int32),
            pltpu.VMEM((2, CHUNK, FEAT), jnp.float32),
            pltpu.VMEM_SHARED((NPAD, FEAT), jnp.float32),
            pltpu.SemaphoreType.DMA,
            pltpu.SemaphoreType.DMA,
            pltpu.SemaphoreType.DMA,
            pltpu.SemaphoreType.DMA,
            pltpu.SemaphoreType.DMA,
            pltpu.SemaphoreType.DMA,
        ],
    )

    ew_specs = [
        pl.BlockSpec((NCORES, CROWS, FEAT), lambda i: (0, i, 0)),
        pl.BlockSpec((CROWS, FEAT), lambda i: (i, 0)),
        pl.BlockSpec((CROWS, FEAT), lambda i: (i, 0)),
    ]
    ew_out = pl.BlockSpec((CROWS, FEAT), lambda i: (i, 0))
    comb = pl.pallas_call(
        _comb_body, grid=(CGRID,), in_specs=ew_specs, out_specs=ew_out,
        out_shape=jax.ShapeDtypeStruct((NPAD, FEAT), jnp.float32))
    fin = pl.pallas_call(
        _fin_body, grid=(CGRID,), in_specs=ew_specs, out_specs=ew_out,
        out_shape=jax.ShapeDtypeStruct((NPAD, FEAT), jnp.float32))

    p = p0
    for _ in range(KHOPS - 1):
        S = scat(p, srcp, dstp)
        p = comb(S, p0, cB)
    S = scat(p, srcp, dstp)
    out = fin(S, p0, cB)
    return out[:N]


# speedup vs baseline: 1.3366x; 1.3366x over previous
import functools

import jax
import jax.numpy as jnp
from jax import lax
from jax.experimental import pallas as pl
from jax.experimental.pallas import tpu as pltpu
from jax.experimental.pallas import tpu_sc as plsc

N = 10000
FEAT = 128
HID = 256
EDGES = 320000
KHOPS = 10
ALPHA = 0.1

NPAD = 10240
NCORES = 2
NSUB = 16
NTILES = NCORES * NSUB
EPT = 10496
EPAD = EPT * NTILES
CHUNK = 128
NCHUNKS = EPT // CHUNK
CGRID = 8
CROWS = NPAD // CGRID
RPT = NPAD // NSUB
GRID = NPAD // 128


def _deg_body(dst_hbm, out_hbm, didx_b, ones_v, zbuf_v, deg_sh,
              dsem0, dsem1):
    c = lax.axis_index("c")
    s = lax.axis_index("s")
    base = (c * NSUB + s) * EPT
    dsems = (dsem0, dsem1)
    for k in range(RPT // 16):
        zbuf_v[pl.ds(k * 16, 16)] = jnp.zeros((16,), jnp.float32)
    for k in range(CHUNK // 16):
        ones_v[pl.ds(k * 16, 16)] = jnp.ones((16,), jnp.float32)
    pltpu.sync_copy(zbuf_v, deg_sh.at[pl.ds(s * RPT, RPT)])
    plsc.subcore_barrier()

    def dfire(j, b):
        pltpu.async_copy(dst_hbm.at[pl.ds(base + j * CHUNK, CHUNK)],
                         didx_b.at[b], dsems[b])

    def dwait(b):
        pltpu.make_async_copy(dst_hbm.at[pl.ds(base, CHUNK)],
                              didx_b.at[b], dsems[b]).wait()

    dfire(0, 0)
    dfire(1, 1)

    def body(i, carry):
        j = 2 * i
        dwait(0)
        pltpu.sync_copy(ones_v, deg_sh.at[didx_b.at[0]], add=True)
        dfire(j + 2, 0)
        dwait(1)
        pltpu.sync_copy(ones_v, deg_sh.at[didx_b.at[1]], add=True)
        dfire(j + 3, 1)
        return carry

    lax.fori_loop(0, NCHUNKS // 2 - 1, body, 0)
    dwait(0)
    pltpu.sync_copy(ones_v, deg_sh.at[didx_b.at[0]], add=True)
    dwait(1)
    pltpu.sync_copy(ones_v, deg_sh.at[didx_b.at[1]], add=True)
    plsc.subcore_barrier()
    pltpu.sync_copy(deg_sh.at[pl.ds(s * RPT, RPT)],
                    out_hbm.at[c, pl.ds(s * RPT, RPT)])


def _scat_body(p_hbm, src_hbm, dst_hbm, out_hbm,
               sidx_all, didx_b, rows_v, s_sh, gsem0, gsem1, dsem0, dsem1):
    c = lax.axis_index("c")
    s = lax.axis_index("s")
    base = (c * NSUB + s) * EPT
    pltpu.sync_copy(src_hbm.at[pl.ds(base, EPT)], sidx_all)
    gsems = (gsem0, gsem1)
    dsems = (dsem0, dsem1)

    def zb(i, carry):
        for k in range(8):
            rows_v[0, i, pl.ds(k * 16, 16)] = jnp.zeros((16,), jnp.float32)
        return carry

    lax.fori_loop(0, 128, zb, 0)
    for t in range(RPT // 128):
        pltpu.sync_copy(rows_v.at[0], s_sh.at[pl.ds(s * RPT + t * 128, 128)])
    plsc.subcore_barrier()

    def dfire(j, b):
        pltpu.async_copy(dst_hbm.at[pl.ds(base + j * CHUNK, CHUNK)],
                         didx_b.at[b], dsems[b])

    def dwait(b):
        pltpu.make_async_copy(dst_hbm.at[pl.ds(base, CHUNK)],
                              didx_b.at[b], dsems[b]).wait()

    def gfire(j, b):
        pltpu.async_copy(p_hbm.at[sidx_all.at[pl.ds(j * CHUNK, CHUNK)]],
                         rows_v.at[b], gsems[b])

    def gwait(b):
        pltpu.make_async_copy(p_hbm.at[sidx_all.at[pl.ds(0, CHUNK)]],
                              rows_v.at[b], gsems[b]).wait()

    def scat(b):
        pltpu.sync_copy(rows_v.at[b], s_sh.at[didx_b.at[b]], add=True)

    dfire(0, 0)
    dfire(1, 1)
    gfire(0, 0)
    gfire(1, 1)

    def body(i, carry):
        j = 2 * i
        gwait(0)
        dwait(0)
        scat(0)
        gfire(j + 2, 0)
        dfire(j + 2, 0)
        gwait(1)
        dwait(1)
        scat(1)
        gfire(j + 3, 1)
        dfire(j + 3, 1)
        return carry

    lax.fori_loop(0, NCHUNKS // 2 - 1, body, 0)
    gwait(0)
    dwait(0)
    scat(0)
    gwait(1)
    dwait(1)
    scat(1)
    plsc.subcore_barrier()
    pltpu.sync_copy(s_sh.at[pl.ds(s * RPT, RPT)],
                    out_hbm.at[c, pl.ds(s * RPT, RPT)])


MROWS = 512
MGRID = NPAD // MROWS


def _mlp_body(xb, w1b, b1b, w2b, b2b, dpb, p0b, cbb):
    h = jnp.maximum(
        jnp.dot(xb[...], w1b[...], preferred_element_type=jnp.float32)
        + b1b[...], 0.0)
    h0 = jnp.dot(h, w2b[...], preferred_element_type=jnp.float32) + b2b[...]
    dp = dpb[...]
    deg = dp[0:1, :] + dp[1:2, :]
    dis = lax.rsqrt(deg)
    dcol = jnp.transpose(dis)
    p0b[...] = h0 * dcol
    cbb[...] = jnp.broadcast_to(0.9 * dcol * dcol, (MROWS, FEAT))


def _comb_body(sb, p0b, cbb, ob):
    sblk = sb[...]
    ob[...] = cbb[...] * (sblk[0] + sblk[1]) + ALPHA * p0b[...]


def _fin_body(sb, p0b, cbb, ob):
    sblk = sb[...]
    cb = cbb[...]
    plast = cb * (sblk[0] + sblk[1]) + ALPHA * p0b[...]
    h = plast * lax.rsqrt(cb * (1.0 / 0.9))
    m = jnp.max(h, axis=1, keepdims=True)
    sh = h - m
    lse = jnp.log(jnp.sum(jnp.exp(sh), axis=1, keepdims=True))
    ob[...] = sh - lse


def kernel(x, edge_index, W1, b1, W2, b2):
    src = edge_index[0]
    dst = edge_index[1]
    loop = jnp.arange(NPAD, dtype=jnp.int32)
    npe = EPAD - EDGES - NPAD
    ar = jnp.arange(npe, dtype=jnp.int32)
    srcp = jnp.concatenate([src, loop, ar % N])
    dstp = jnp.concatenate([dst, loop, N + ar % (NPAD - N)])
    xp = jnp.pad(x, ((0, NPAD - N), (0, 0)))
    b1r = b1.reshape(1, HID)
    b2r = b2.reshape(1, FEAT)

    mesh = plsc.VectorSubcoreMesh(core_axis_name="c", subcore_axis_name="s")

    deg_call = pl.kernel(
        _deg_body,
        out_type=jax.ShapeDtypeStruct((NCORES, NPAD), jnp.float32),
        mesh=mesh,
        scratch_types=[
            pltpu.VMEM((2, CHUNK), jnp.int32),
            pltpu.VMEM((CHUNK,), jnp.float32),
            pltpu.VMEM((RPT,), jnp.float32),
            pltpu.VMEM_SHARED((NPAD,), jnp.float32),
            pltpu.SemaphoreType.DMA,
            pltpu.SemaphoreType.DMA,
        ],
    )
    degp = deg_call(dstp)

    mlp = pl.pallas_call(
        _mlp_body,
        grid=(MGRID,),
        in_specs=[
            pl.BlockSpec((MROWS, FEAT), lambda i: (i, 0)),
            pl.BlockSpec((FEAT, HID), lambda i: (0, 0)),
            pl.BlockSpec((1, HID), lambda i: (0, 0)),
            pl.BlockSpec((HID, FEAT), lambda i: (0, 0)),
            pl.BlockSpec((1, FEAT), lambda i: (0, 0)),
            pl.BlockSpec((NCORES, MROWS), lambda i: (0, i)),
        ],
        out_specs=[
            pl.BlockSpec((MROWS, FEAT), lambda i: (i, 0)),
            pl.BlockSpec((MROWS, FEAT), lambda i: (i, 0)),
        ],
        out_shape=[
            jax.ShapeDtypeStruct((NPAD, FEAT), jnp.float32),
            jax.ShapeDtypeStruct((NPAD, FEAT), jnp.float32),
        ],
    )
    p0, cB = mlp(xp, W1, b1r, W2, b2r, degp)

    scat = pl.kernel(
        _scat_body,
        out_type=jax.ShapeDtypeStruct((NCORES, NPAD, FEAT), jnp.float32),
        mesh=mesh,
        scratch_types=[
            pltpu.VMEM((EPT,), jnp.int32),
            pltpu.VMEM((2, CHUNK), jnp.int32),
            pltpu.VMEM((2, CHUNK, FEAT), jnp.float32),
            pltpu.VMEM_SHARED((NPAD, FEAT), jnp.float32),
            pltpu.SemaphoreType.DMA,
            pltpu.SemaphoreType.DMA,
            pltpu.SemaphoreType.DMA,
            pltpu.SemaphoreType.DMA,
        ],
    )

    ew_specs = [
        pl.BlockSpec((NCORES, CROWS, FEAT), lambda i: (0, i, 0)),
        pl.BlockSpec((CROWS, FEAT), lambda i: (i, 0)),
        pl.BlockSpec((CROWS, FEAT), lambda i: (i, 0)),
    ]
    ew_out = pl.BlockSpec((CROWS, FEAT), lambda i: (i, 0))
    comb = pl.pallas_call(
        _comb_body, grid=(CGRID,), in_specs=ew_specs, out_specs=ew_out,
        out_shape=jax.ShapeDtypeStruct((NPAD, FEAT), jnp.float32))
    fin = pl.pallas_call(
        _fin_body, grid=(CGRID,), in_specs=ew_specs, out_specs=ew_out,
        out_shape=jax.ShapeDtypeStruct((NPAD, FEAT), jnp.float32))

    p = p0
    for _ in range(KHOPS - 1):
        S = scat(p, srcp, dstp)
        p = comb(S, p0, cB)
    S = scat(p, srcp, dstp)
    out = fin(S, p0, cB)
    return out[:N]


# speedup vs baseline: 1.3438x; 1.0054x over previous
import functools

import jax
import jax.numpy as jnp
from jax import lax
from jax.experimental import pallas as pl
from jax.experimental.pallas import tpu as pltpu
from jax.experimental.pallas import tpu_sc as plsc

N = 10000
FEAT = 128
HID = 256
EDGES = 320000
KHOPS = 10
ALPHA = 0.1

NPAD = 10240
NCORES = 2
NSUB = 16
NTILES = NCORES * NSUB
EPT = 10496
EPAD = EPT * NTILES
CHUNK = 128
NCHUNKS = EPT // CHUNK
CGRID = 8
CROWS = NPAD // CGRID
RPT = NPAD // NSUB
GRID = NPAD // 128


def _deg_body(dst_hbm, out_hbm, didx_b, ones_v, zbuf_v, deg_sh,
              dsem0, dsem1):
    c = lax.axis_index("c")
    s = lax.axis_index("s")
    base = (c * NSUB + s) * EPT
    dsems = (dsem0, dsem1)
    for k in range(RPT // 16):
        zbuf_v[pl.ds(k * 16, 16)] = jnp.zeros((16,), jnp.float32)
    for k in range(CHUNK // 16):
        ones_v[pl.ds(k * 16, 16)] = jnp.ones((16,), jnp.float32)
    pltpu.sync_copy(zbuf_v, deg_sh.at[pl.ds(s * RPT, RPT)])
    plsc.subcore_barrier()

    def dfire(j, b):
        pltpu.async_copy(dst_hbm.at[pl.ds(base + j * CHUNK, CHUNK)],
                         didx_b.at[b], dsems[b])

    def dwait(b):
        pltpu.make_async_copy(dst_hbm.at[pl.ds(base, CHUNK)],
                              didx_b.at[b], dsems[b]).wait()

    dfire(0, 0)
    dfire(1, 1)

    def body(i, carry):
        j = 2 * i
        dwait(0)
        pltpu.sync_copy(ones_v, deg_sh.at[didx_b.at[0]], add=True)
        dfire(j + 2, 0)
        dwait(1)
        pltpu.sync_copy(ones_v, deg_sh.at[didx_b.at[1]], add=True)
        dfire(j + 3, 1)
        return carry

    lax.fori_loop(0, NCHUNKS // 2 - 1, body, 0)
    dwait(0)
    pltpu.sync_copy(ones_v, deg_sh.at[didx_b.at[0]], add=True)
    dwait(1)
    pltpu.sync_copy(ones_v, deg_sh.at[didx_b.at[1]], add=True)
    plsc.subcore_barrier()
    pltpu.sync_copy(deg_sh.at[pl.ds(s * RPT, RPT)],
                    out_hbm.at[c, pl.ds(s * RPT, RPT)])


def _scat_body(p_hbm, src_hbm, dst_hbm, out_hbm,
               sidx_all, didx_b, rows_v, s_sh, gsem0, gsem1, dsem0, dsem1):
    c = lax.axis_index("c")
    s = lax.axis_index("s")
    base = (c * NSUB + s) * EPT
    pltpu.sync_copy(src_hbm.at[pl.ds(base, EPT)], sidx_all)
    gsems = (gsem0, gsem1)
    dsems = (dsem0, dsem1)

    def zb(i, carry):
        for k in range(8):
            rows_v[0, i, pl.ds(k * 16, 16)] = jnp.zeros((16,), jnp.float32)
        return carry

    lax.fori_loop(0, 128, zb, 0)
    for t in range(RPT // 128):
        pltpu.sync_copy(rows_v.at[0], s_sh.at[pl.ds(s * RPT + t * 128, 128)])
    plsc.subcore_barrier()

    def dfire(j, b):
        pltpu.async_copy(dst_hbm.at[pl.ds(base + j * CHUNK, CHUNK)],
                         didx_b.at[b], dsems[b])

    def dwait(b):
        pltpu.make_async_copy(dst_hbm.at[pl.ds(base, CHUNK)],
                              didx_b.at[b], dsems[b]).wait()

    def gfire(j, b):
        pltpu.async_copy(p_hbm.at[sidx_all.at[pl.ds(j * CHUNK, CHUNK)]],
                         rows_v.at[b], gsems[b])

    def gwait(b):
        pltpu.make_async_copy(p_hbm.at[sidx_all.at[pl.ds(0, CHUNK)]],
                              rows_v.at[b], gsems[b]).wait()

    def scat(b):
        pltpu.sync_copy(rows_v.at[b], s_sh.at[didx_b.at[b]], add=True)

    dfire(0, 0)
    dfire(1, 1)
    gfire(0, 0)
    gfire(1, 1)

    def body(i, carry):
        j = 2 * i
        gwait(0)
        dwait(0)
        scat(0)
        gfire(j + 2, 0)
        dfire(j + 2, 0)
        gwait(1)
        dwait(1)
        scat(1)
        gfire(j + 3, 1)
        dfire(j + 3, 1)
        return carry

    lax.fori_loop(0, NCHUNKS // 2 - 1, body, 0)
    gwait(0)
    dwait(0)
    scat(0)
    gwait(1)
    dwait(1)
    scat(1)
    plsc.subcore_barrier()
    pltpu.sync_copy(s_sh.at[pl.ds(s * RPT, RPT)],
                    out_hbm.at[c, pl.ds(s * RPT, RPT)])


MROWS = NPAD
MGRID = NPAD // MROWS


def _mlp_body(xb, w1b, b1b, w2b, b2b, dpb, p0b, cbb):
    h = jnp.maximum(
        jnp.dot(xb[...], w1b[...], preferred_element_type=jnp.float32)
        + b1b[...], 0.0)
    h0 = jnp.dot(h, w2b[...], preferred_element_type=jnp.float32) + b2b[...]
    dp = dpb[...]
    deg = dp[0:1, :] + dp[1:2, :]
    dis = lax.rsqrt(deg)
    dcol = jnp.transpose(dis)
    p0b[...] = h0 * dcol
    cbb[...] = jnp.broadcast_to(0.9 * dcol * dcol, (MROWS, FEAT))


def _comb_body(sb, p0b, cbb, ob):
    sblk = sb[...]
    ob[...] = cbb[...] * (sblk[0] + sblk[1]) + ALPHA * p0b[...]


def _fin_body(sb, p0b, cbb, ob):
    sblk = sb[...]
    cb = cbb[...]
    plast = cb * (sblk[0] + sblk[1]) + ALPHA * p0b[...]
    h = plast * lax.rsqrt(cb * (1.0 / 0.9))
    m = jnp.max(h, axis=1, keepdims=True)
    sh = h - m
    lse = jnp.log(jnp.sum(jnp.exp(sh), axis=1, keepdims=True))
    ob[...] = sh - lse


def kernel(x, edge_index, W1, b1, W2, b2):
    src = edge_index[0]
    dst = edge_index[1]
    loop = jnp.arange(NPAD, dtype=jnp.int32)
    npe = EPAD - EDGES - NPAD
    ar = jnp.arange(npe, dtype=jnp.int32)
    srcp = jnp.concatenate([src, loop, ar % N])
    dstp = jnp.concatenate([dst, loop, N + ar % (NPAD - N)])
    xp = jnp.pad(x, ((0, NPAD - N), (0, 0)))
    b1r = b1.reshape(1, HID)
    b2r = b2.reshape(1, FEAT)

    mesh = plsc.VectorSubcoreMesh(core_axis_name="c", subcore_axis_name="s")

    deg_call = pl.kernel(
        _deg_body,
        out_type=jax.ShapeDtypeStruct((NCORES, NPAD), jnp.float32),
        mesh=mesh,
        scratch_types=[
            pltpu.VMEM((2, CHUNK), jnp.int32),
            pltpu.VMEM((CHUNK,), jnp.float32),
            pltpu.VMEM((RPT,), jnp.float32),
            pltpu.VMEM_SHARED((NPAD,), jnp.float32),
            pltpu.SemaphoreType.DMA,
            pltpu.SemaphoreType.DMA,
        ],
    )
    degp = deg_call(dstp)

    mlp = pl.pallas_call(
        _mlp_body,
        grid=(MGRID,),
        in_specs=[
            pl.BlockSpec((MROWS, FEAT), lambda i: (i, 0)),
            pl.BlockSpec((FEAT, HID), lambda i: (0, 0)),
            pl.BlockSpec((1, HID), lambda i: (0, 0)),
            pl.BlockSpec((HID, FEAT), lambda i: (0, 0)),
            pl.BlockSpec((1, FEAT), lambda i: (0, 0)),
            pl.BlockSpec((NCORES, MROWS), lambda i: (0, i)),
        ],
        out_specs=[
            pl.BlockSpec((MROWS, FEAT), lambda i: (i, 0)),
            pl.BlockSpec((MROWS, FEAT), lambda i: (i, 0)),
        ],
        out_shape=[
            jax.ShapeDtypeStruct((NPAD, FEAT), jnp.float32),
            jax.ShapeDtypeStruct((NPAD, FEAT), jnp.float32),
        ],
    )
    p0, cB = mlp(xp, W1, b1r, W2, b2r, degp)

    scat = pl.kernel(
        _scat_body,
        out_type=jax.ShapeDtypeStruct((NCORES, NPAD, FEAT), jnp.float32),
        mesh=mesh,
        scratch_types=[
            pltpu.VMEM((EPT,), jnp.int32),
            pltpu.VMEM((2, CHUNK), jnp.int32),
            pltpu.VMEM((2, CHUNK, FEAT), jnp.float32),
            pltpu.VMEM_SHARED((NPAD, FEAT), jnp.float32),
            pltpu.SemaphoreType.DMA,
            pltpu.SemaphoreType.DMA,
            pltpu.SemaphoreType.DMA,
            pltpu.SemaphoreType.DMA,
        ],
    )

    ew_specs = [
        pl.BlockSpec((NCORES, CROWS, FEAT), lambda i: (0, i, 0)),
        pl.BlockSpec((CROWS, FEAT), lambda i: (i, 0)),
        pl.BlockSpec((CROWS, FEAT), lambda i: (i, 0)),
    ]
    ew_out = pl.BlockSpec((CROWS, FEAT), lambda i: (i, 0))
    comb = pl.pallas_call(
        _comb_body, grid=(CGRID,), in_specs=ew_specs, out_specs=ew_out,
        out_shape=jax.ShapeDtypeStruct((NPAD, FEAT), jnp.float32))
    fin = pl.pallas_call(
        _fin_body, grid=(CGRID,), in_specs=ew_specs, out_specs=ew_out,
        out_shape=jax.ShapeDtypeStruct((NPAD, FEAT), jnp.float32))

    p = p0
    for _ in range(KHOPS - 1):
        S = scat(p, srcp, dstp)
        p = comb(S, p0, cB)
    S = scat(p, srcp, dstp)
    out = fin(S, p0, cB)
    return out[:N]


# speedup vs baseline: 1.5016x; 1.1174x over previous
import functools

import jax
import jax.numpy as jnp
from jax import lax
from jax.experimental import pallas as pl
from jax.experimental.pallas import tpu as pltpu
from jax.experimental.pallas import tpu_sc as plsc

N = 10000
FEAT = 128
HID = 256
EDGES = 320000
KHOPS = 10
ALPHA = 0.1

NPAD = 10112
NCORES = 2
NSUB = 16
NTILES = NCORES * NSUB
EPT = 10368
EPAD = EPT * NTILES
CHUNK = 96
NCHUNKS = EPT // CHUNK
NBUF = 3
CGRID = 8
CROWS = NPAD // CGRID
RPT = NPAD // NSUB


def _deg_body(dst_hbm, out_hbm, didx_b, ones_v, zbuf_v, deg_sh,
              dsem0, dsem1):
    c = lax.axis_index("c")
    s = lax.axis_index("s")
    base = (c * NSUB + s) * EPT
    dsems = (dsem0, dsem1)
    for k in range(40):
        zbuf_v[pl.ds(k * 16, 16)] = jnp.zeros((16,), jnp.float32)
    for k in range(CHUNK // 16):
        ones_v[pl.ds(k * 16, 16)] = jnp.ones((16,), jnp.float32)

    @pl.when(s < NSUB - 1)
    def _():
        pltpu.sync_copy(zbuf_v, deg_sh.at[pl.ds(s * 640, 640)])

    @pl.when(s == NSUB - 1)
    def _():
        pltpu.sync_copy(zbuf_v.at[pl.ds(0, 512)], deg_sh.at[pl.ds(9600, 512)])

    plsc.subcore_barrier()

    def dfire(j, b):
        pltpu.async_copy(dst_hbm.at[pl.ds(base + j * CHUNK, CHUNK)],
                         didx_b.at[b], dsems[b])

    def dwait(b):
        pltpu.make_async_copy(dst_hbm.at[pl.ds(base, CHUNK)],
                              didx_b.at[b], dsems[b]).wait()

    dfire(0, 0)
    dfire(1, 1)

    def body(i, carry):
        j = 2 * i
        dwait(0)
        pltpu.sync_copy(ones_v, deg_sh.at[didx_b.at[0]], add=True)
        dfire(j + 2, 0)
        dwait(1)
        pltpu.sync_copy(ones_v, deg_sh.at[didx_b.at[1]], add=True)
        dfire(j + 3, 1)
        return carry

    lax.fori_loop(0, NCHUNKS // 2 - 1, body, 0)
    dwait(0)
    pltpu.sync_copy(ones_v, deg_sh.at[didx_b.at[0]], add=True)
    dwait(1)
    pltpu.sync_copy(ones_v, deg_sh.at[didx_b.at[1]], add=True)
    plsc.subcore_barrier()

    @pl.when(s < NSUB - 1)
    def _():
        pltpu.sync_copy(deg_sh.at[pl.ds(s * 640, 640)],
                        out_hbm.at[pl.ds(c * NPAD + s * 640, 640)])

    @pl.when(s == NSUB - 1)
    def _():
        pltpu.sync_copy(deg_sh.at[pl.ds(9600, 512)],
                        out_hbm.at[pl.ds(c * NPAD + 9600, 512)])


def _scat_body(p_hbm, src_hbm, dst_hbm, out_hbm,
               sidx_all, didx_b, rows_v, s_sh,
               gsem0, gsem1, gsem2, dsem0, dsem1, dsem2):
    c = lax.axis_index("c")
    s = lax.axis_index("s")
    base = (c * NSUB + s) * EPT
    pltpu.sync_copy(src_hbm.at[pl.ds(base, EPT)], sidx_all)
    gsems = (gsem0, gsem1, gsem2)
    dsems = (dsem0, dsem1, dsem2)

    def zb(i, carry):
        for k in range(8):
            rows_v[0, i, pl.ds(k * 16, 16)] = jnp.zeros((16,), jnp.float32)
        return carry

    lax.fori_loop(0, CHUNK, zb, 0)
    for t in range(RPT // CHUNK):
        pltpu.sync_copy(rows_v.at[0],
                        s_sh.at[pl.ds(s * RPT + t * CHUNK, CHUNK)])
    rem = RPT % CHUNK
    pltpu.sync_copy(rows_v.at[0, pl.ds(0, rem)],
                    s_sh.at[pl.ds(s * RPT + RPT - rem, rem)])
    plsc.subcore_barrier()

    def dfire(j, b):
        pltpu.async_copy(dst_hbm.at[pl.ds(base + j * CHUNK, CHUNK)],
                         didx_b.at[b], dsems[b])

    def dwait(b):
        pltpu.make_async_copy(dst_hbm.at[pl.ds(base, CHUNK)],
                              didx_b.at[b], dsems[b]).wait()

    def gfire(j, b):
        pltpu.async_copy(p_hbm.at[sidx_all.at[pl.ds(j * CHUNK, CHUNK)]],
                         rows_v.at[b], gsems[b])

    def gwait(b):
        pltpu.make_async_copy(p_hbm.at[sidx_all.at[pl.ds(0, CHUNK)]],
                              rows_v.at[b], gsems[b]).wait()

    def scat(b):
        pltpu.sync_copy(rows_v.at[b], s_sh.at[didx_b.at[b]], add=True)

    for b in range(NBUF):
        dfire(b, b)
        gfire(b, b)

    def body(i, carry):
        j = 3 * i
        for b in range(NBUF):
            gwait(b)
            dwait(b)
            scat(b)
            gfire(j + b + 3, b)
            dfire(j + b + 3, b)
        return carry

    lax.fori_loop(0, NCHUNKS // 3 - 1, body, 0)
    for b in range(NBUF):
        gwait(b)
        dwait(b)
        scat(b)
    plsc.subcore_barrier()
    pltpu.sync_copy(s_sh.at[pl.ds(s * RPT, RPT)],
                    out_hbm.at[c, pl.ds(s * RPT, RPT)])


MROWS = NPAD
MGRID = NPAD // MROWS


def _mlp_body(xb, w1b, b1b, w2b, b2b, dpb, p0b, cbb):
    h = jnp.maximum(
        jnp.dot(xb[...], w1b[...], preferred_element_type=jnp.float32)
        + b1b[...], 0.0)
    h0 = jnp.dot(h, w2b[...], preferred_element_type=jnp.float32) + b2b[...]
    dp = dpb[...]
    deg = dp[0:1, :] + dp[1:2, :]
    dis = lax.rsqrt(deg)
    dcol = jnp.transpose(dis)
    p0b[...] = h0 * dcol
    cbb[...] = jnp.broadcast_to(0.9 * dcol * dcol, (MROWS, FEAT))


def _comb_body(sb, p0b, cbb, ob):
    sblk = sb[...]
    ob[...] = cbb[...] * (sblk[0] + sblk[1]) + ALPHA * p0b[...]


def _fin_body(sb, p0b, cbb, ob):
    sblk = sb[...]
    cb = cbb[...]
    plast = cb * (sblk[0] + sblk[1]) + ALPHA * p0b[...]
    h = plast * lax.rsqrt(cb * (1.0 / 0.9))
    m = jnp.max(h, axis=1, keepdims=True)
    sh = h - m
    lse = jnp.log(jnp.sum(jnp.exp(sh), axis=1, keepdims=True))
    ob[...] = sh - lse


def kernel(x, edge_index, W1, b1, W2, b2):
    src = edge_index[0]
    dst = edge_index[1]
    loop = jnp.arange(NPAD, dtype=jnp.int32)
    npe = EPAD - EDGES - NPAD
    ar = jnp.arange(npe, dtype=jnp.int32)
    srcp = jnp.concatenate([src, loop, ar % N])
    dstp = jnp.concatenate([dst, loop, N + ar % (NPAD - N)])
    xp = jnp.pad(x, ((0, NPAD - N), (0, 0)))
    b1r = b1.reshape(1, HID)
    b2r = b2.reshape(1, FEAT)

    mesh = plsc.VectorSubcoreMesh(core_axis_name="c", subcore_axis_name="s")

    deg_call = pl.kernel(
        _deg_body,
        out_type=jax.ShapeDtypeStruct((NCORES * NPAD,), jnp.float32),
        mesh=mesh,
        scratch_types=[
            pltpu.VMEM((2, CHUNK), jnp.int32),
            pltpu.VMEM((CHUNK,), jnp.float32),
            pltpu.VMEM((640,), jnp.float32),
            pltpu.VMEM_SHARED((NPAD,), jnp.float32),
            pltpu.SemaphoreType.DMA,
            pltpu.SemaphoreType.DMA,
        ],
    )
    degp = deg_call(dstp).reshape(NCORES, NPAD)

    mlp = pl.pallas_call(
        _mlp_body,
        grid=(MGRID,),
        in_specs=[
            pl.BlockSpec((MROWS, FEAT), lambda i: (i, 0)),
            pl.BlockSpec((FEAT, HID), lambda i: (0, 0)),
            pl.BlockSpec((1, HID), lambda i: (0, 0)),
            pl.BlockSpec((HID, FEAT), lambda i: (0, 0)),
            pl.BlockSpec((1, FEAT), lambda i: (0, 0)),
            pl.BlockSpec((NCORES, MROWS), lambda i: (0, i)),
        ],
        out_specs=[
            pl.BlockSpec((MROWS, FEAT), lambda i: (i, 0)),
            pl.BlockSpec((MROWS, FEAT), lambda i: (i, 0)),
        ],
        out_shape=[
            jax.ShapeDtypeStruct((NPAD, FEAT), jnp.float32),
            jax.ShapeDtypeStruct((NPAD, FEAT), jnp.float32),
        ],
    )
    p0, cB = mlp(xp, W1, b1r, W2, b2r, degp)

    scat = pl.kernel(
        _scat_body,
        out_type=jax.ShapeDtypeStruct((NCORES, NPAD, FEAT), jnp.float32),
        mesh=mesh,
        scratch_types=[
            pltpu.VMEM((EPT,), jnp.int32),
            pltpu.VMEM((NBUF, CHUNK), jnp.int32),
            pltpu.VMEM((NBUF, CHUNK, FEAT), jnp.float32),
            pltpu.VMEM_SHARED((NPAD, FEAT), jnp.float32),
            pltpu.SemaphoreType.DMA,
            pltpu.SemaphoreType.DMA,
            pltpu.SemaphoreType.DMA,
            pltpu.SemaphoreType.DMA,
            pltpu.SemaphoreType.DMA,
            pltpu.SemaphoreType.DMA,
        ],
    )

    ew_specs = [
        pl.BlockSpec((NCORES, CROWS, FEAT), lambda i: (0, i, 0)),
        pl.BlockSpec((CROWS, FEAT), lambda i: (i, 0)),
        pl.BlockSpec((CROWS, FEAT), lambda i: (i, 0)),
    ]
    ew_out = pl.BlockSpec((CROWS, FEAT), lambda i: (i, 0))
    comb = pl.pallas_call(
        _comb_body, grid=(CGRID,), in_specs=ew_specs, out_specs=ew_out,
        out_shape=jax.ShapeDtypeStruct((NPAD, FEAT), jnp.float32))
    fin = pl.pallas_call(
        _fin_body, grid=(CGRID,), in_specs=ew_specs, out_specs=ew_out,
        out_shape=jax.ShapeDtypeStruct((NPAD, FEAT), jnp.float32))

    p = p0
    for _ in range(KHOPS - 1):
        S = scat(p, srcp, dstp)
        p = comb(S, p0, cB)
    S = scat(p, srcp, dstp)
    out = fin(S, p0, cB)
    return out[:N]


# speedup vs baseline: 1.5159x; 1.0095x over previous
import functools

import jax
import jax.numpy as jnp
from jax import lax
from jax.experimental import pallas as pl
from jax.experimental.pallas import tpu as pltpu
from jax.experimental.pallas import tpu_sc as plsc

N = 10000
FEAT = 128
HID = 256
EDGES = 320000
KHOPS = 10
ALPHA = 0.1

NPAD = 10112
NCORES = 2
NSUB = 16
NTILES = NCORES * NSUB
EPT = 10368
EPAD = EPT * NTILES
CHUNK = 96
NCHUNKS = EPT // CHUNK
NBUF = 3
CGRID = 8
CROWS = NPAD // CGRID
RPT = NPAD // NSUB


def _deg_body(dst_hbm, out_hbm, didx_b, ones_v, zbuf_v, deg_sh,
              dsem0, dsem1):
    c = lax.axis_index("c")
    s = lax.axis_index("s")
    base = (c * NSUB + s) * EPT
    dsems = (dsem0, dsem1)
    for k in range(40):
        zbuf_v[pl.ds(k * 16, 16)] = jnp.zeros((16,), jnp.float32)
    for k in range(CHUNK // 16):
        ones_v[pl.ds(k * 16, 16)] = jnp.ones((16,), jnp.float32)

    @pl.when(s < NSUB - 1)
    def _():
        pltpu.sync_copy(zbuf_v, deg_sh.at[pl.ds(s * 640, 640)])

    @pl.when(s == NSUB - 1)
    def _():
        pltpu.sync_copy(zbuf_v.at[pl.ds(0, 512)], deg_sh.at[pl.ds(9600, 512)])

    plsc.subcore_barrier()

    def dfire(j, b):
        pltpu.async_copy(dst_hbm.at[pl.ds(base + j * CHUNK, CHUNK)],
                         didx_b.at[b], dsems[b])

    def dwait(b):
        pltpu.make_async_copy(dst_hbm.at[pl.ds(base, CHUNK)],
                              didx_b.at[b], dsems[b]).wait()

    dfire(0, 0)
    dfire(1, 1)

    def body(i, carry):
        j = 2 * i
        dwait(0)
        pltpu.sync_copy(ones_v, deg_sh.at[didx_b.at[0]], add=True)
        dfire(j + 2, 0)
        dwait(1)
        pltpu.sync_copy(ones_v, deg_sh.at[didx_b.at[1]], add=True)
        dfire(j + 3, 1)
        return carry

    lax.fori_loop(0, NCHUNKS // 2 - 1, body, 0)
    dwait(0)
    pltpu.sync_copy(ones_v, deg_sh.at[didx_b.at[0]], add=True)
    dwait(1)
    pltpu.sync_copy(ones_v, deg_sh.at[didx_b.at[1]], add=True)
    plsc.subcore_barrier()

    @pl.when(s < NSUB - 1)
    def _():
        pltpu.sync_copy(deg_sh.at[pl.ds(s * 640, 640)],
                        out_hbm.at[pl.ds(c * NPAD + s * 640, 640)])

    @pl.when(s == NSUB - 1)
    def _():
        pltpu.sync_copy(deg_sh.at[pl.ds(9600, 512)],
                        out_hbm.at[pl.ds(c * NPAD + 9600, 512)])


def _scat_body(p_hbm, src_hbm, dst_hbm, out_hbm,
               sidx_all, didx_b, rows_v, s_sh,
               gsem0, gsem1, gsem2, dsem0, dsem1, dsem2):
    c = lax.axis_index("c")
    s = lax.axis_index("s")
    base = (c * NSUB + s) * EPT
    pltpu.sync_copy(src_hbm.at[pl.ds(base, EPT)], sidx_all)
    gsems = (gsem0, gsem1, gsem2)
    dsems = (dsem0, dsem1, dsem2)

    def dfire(j, b):
        pltpu.async_copy(dst_hbm.at[pl.ds(base + j * CHUNK, CHUNK)],
                         didx_b.at[b], dsems[b])

    def dwait(b):
        pltpu.make_async_copy(dst_hbm.at[pl.ds(base, CHUNK)],
                              didx_b.at[b], dsems[b]).wait()

    def gfire(j, b):
        pltpu.async_copy(p_hbm.at[sidx_all.at[pl.ds(j * CHUNK, CHUNK)]],
                         rows_v.at[b], gsems[b])

    def gwait(b):
        pltpu.make_async_copy(p_hbm.at[sidx_all.at[pl.ds(0, CHUNK)]],
                              rows_v.at[b], gsems[b]).wait()

    def scat(b):
        pltpu.sync_copy(rows_v.at[b], s_sh.at[didx_b.at[b]], add=True)

    dfire(1, 1)
    gfire(1, 1)
    dfire(2, 2)
    gfire(2, 2)

    def zb(i, carry):
        for k in range(8):
            rows_v[0, i, pl.ds(k * 16, 16)] = jnp.zeros((16,), jnp.float32)
        return carry

    lax.fori_loop(0, CHUNK, zb, 0)
    for t in range(RPT // CHUNK):
        pltpu.sync_copy(rows_v.at[0],
                        s_sh.at[pl.ds(s * RPT + t * CHUNK, CHUNK)])
    rem = RPT % CHUNK
    pltpu.sync_copy(rows_v.at[0, pl.ds(0, rem)],
                    s_sh.at[pl.ds(s * RPT + RPT - rem, rem)])
    plsc.subcore_barrier()
    dfire(0, 0)
    gfire(0, 0)

    def body(i, carry):
        j = 3 * i
        for b in range(NBUF):
            gwait(b)
            dwait(b)
            scat(b)
            gfire(j + b + 3, b)
            dfire(j + b + 3, b)
        return carry

    lax.fori_loop(0, NCHUNKS // 3 - 1, body, 0)
    for b in range(NBUF):
        gwait(b)
        dwait(b)
        scat(b)
    plsc.subcore_barrier()
    pltpu.sync_copy(s_sh.at[pl.ds(s * RPT, RPT)],
                    out_hbm.at[c, pl.ds(s * RPT, RPT)])


MROWS = NPAD
MGRID = NPAD // MROWS


def _mlp_body(xb, w1b, b1b, w2b, b2b, dpb, p0b, cbb):
    h = jnp.maximum(
        jnp.dot(xb[...], w1b[...], preferred_element_type=jnp.float32)
        + b1b[...], 0.0)
    h0 = jnp.dot(h, w2b[...], preferred_element_type=jnp.float32) + b2b[...]
    dp = dpb[...]
    deg = dp[0:1, :] + dp[1:2, :]
    dis = lax.rsqrt(deg)
    dcol = jnp.transpose(dis)
    p0b[...] = h0 * dcol
    cbb[...] = jnp.broadcast_to(0.9 * dcol * dcol, (MROWS, FEAT))


def _comb_body(sb, p0b, cbb, ob):
    sblk = sb[...]
    ob[...] = cbb[...] * (sblk[0] + sblk[1]) + ALPHA * p0b[...]


def _fin_body(sb, p0b, cbb, ob):
    sblk = sb[...]
    cb = cbb[...]
    plast = cb * (sblk[0] + sblk[1]) + ALPHA * p0b[...]
    h = plast * lax.rsqrt(cb * (1.0 / 0.9))
    m = jnp.max(h, axis=1, keepdims=True)
    sh = h - m
    lse = jnp.log(jnp.sum(jnp.exp(sh), axis=1, keepdims=True))
    ob[...] = sh - lse


def kernel(x, edge_index, W1, b1, W2, b2):
    src = edge_index[0]
    dst = edge_index[1]
    loop = jnp.arange(NPAD, dtype=jnp.int32)
    npe = EPAD - EDGES - NPAD
    ar = jnp.arange(npe, dtype=jnp.int32)
    srcp = jnp.concatenate([src, loop, ar % N])
    dstp = jnp.concatenate([dst, loop, N + ar % (NPAD - N)])
    xp = jnp.pad(x, ((0, NPAD - N), (0, 0)))
    b1r = b1.reshape(1, HID)
    b2r = b2.reshape(1, FEAT)

    mesh = plsc.VectorSubcoreMesh(core_axis_name="c", subcore_axis_name="s")

    deg_call = pl.kernel(
        _deg_body,
        out_type=jax.ShapeDtypeStruct((NCORES * NPAD,), jnp.float32),
        mesh=mesh,
        scratch_types=[
            pltpu.VMEM((2, CHUNK), jnp.int32),
            pltpu.VMEM((CHUNK,), jnp.float32),
            pltpu.VMEM((640,), jnp.float32),
            pltpu.VMEM_SHARED((NPAD,), jnp.float32),
            pltpu.SemaphoreType.DMA,
            pltpu.SemaphoreType.DMA,
        ],
    )
    degp = deg_call(dstp).reshape(NCORES, NPAD)

    mlp = pl.pallas_call(
        _mlp_body,
        grid=(MGRID,),
        in_specs=[
            pl.BlockSpec((MROWS, FEAT), lambda i: (i, 0)),
            pl.BlockSpec((FEAT, HID), lambda i: (0, 0)),
            pl.BlockSpec((1, HID), lambda i: (0, 0)),
            pl.BlockSpec((HID, FEAT), lambda i: (0, 0)),
            pl.BlockSpec((1, FEAT), lambda i: (0, 0)),
            pl.BlockSpec((NCORES, MROWS), lambda i: (0, i)),
        ],
        out_specs=[
            pl.BlockSpec((MROWS, FEAT), lambda i: (i, 0)),
            pl.BlockSpec((MROWS, FEAT), lambda i: (i, 0)),
        ],
        out_shape=[
            jax.ShapeDtypeStruct((NPAD, FEAT), jnp.float32),
            jax.ShapeDtypeStruct((NPAD, FEAT), jnp.float32),
        ],
    )
    p0, cB = mlp(xp, W1, b1r, W2, b2r, degp)

    scat = pl.kernel(
        _scat_body,
        out_type=jax.ShapeDtypeStruct((NCORES, NPAD, FEAT), jnp.float32),
        mesh=mesh,
        scratch_types=[
            pltpu.VMEM((EPT,), jnp.int32),
            pltpu.VMEM((NBUF, CHUNK), jnp.int32),
            pltpu.VMEM((NBUF, CHUNK, FEAT), jnp.float32),
            pltpu.VMEM_SHARED((NPAD, FEAT), jnp.float32),
            pltpu.SemaphoreType.DMA,
            pltpu.SemaphoreType.DMA,
            pltpu.SemaphoreType.DMA,
            pltpu.SemaphoreType.DMA,
            pltpu.SemaphoreType.DMA,
            pltpu.SemaphoreType.DMA,
        ],
    )

    ew_specs = [
        pl.BlockSpec((NCORES, CROWS, FEAT), lambda i: (0, i, 0)),
        pl.BlockSpec((CROWS, FEAT), lambda i: (i, 0)),
        pl.BlockSpec((CROWS, FEAT), lambda i: (i, 0)),
    ]
    ew_out = pl.BlockSpec((CROWS, FEAT), lambda i: (i, 0))
    comb = pl.pallas_call(
        _comb_body, grid=(CGRID,), in_specs=ew_specs, out_specs=ew_out,
        out_shape=jax.ShapeDtypeStruct((NPAD, FEAT), jnp.float32))
    fin = pl.pallas_call(
        _fin_body, grid=(CGRID,), in_specs=ew_specs, out_specs=ew_out,
        out_shape=jax.ShapeDtypeStruct((N, FEAT), jnp.float32))

    p = p0
    for _ in range(KHOPS - 1):
        S = scat(p, srcp, dstp)
        p = comb(S, p0, cB)
    S = scat(p, srcp, dstp)
    return fin(S, p0, cB)


# speedup vs baseline: 1.5171x; 1.0008x over previous
import jax
import jax.numpy as jnp
from jax import lax
from jax.experimental import pallas as pl
from jax.experimental.pallas import tpu as pltpu
from jax.experimental.pallas import tpu_sc as plsc

N = 10000
FEAT = 128
HID = 256
EDGES = 320000
KHOPS = 10
ALPHA = 0.1

NPAD = 10112
NCORES = 2
NSUB = 16
NTILES = NCORES * NSUB
EPT = 10368
EPAD = EPT * NTILES
CHUNK = 96
NCHUNKS = EPT // CHUNK
NBUF = 3
CGRID = 8
CROWS = NPAD // CGRID
RPT = NPAD // NSUB


def _deg_body(dst_hbm, out_hbm, didx_b, ones_v, zbuf_v, deg_sh,
              dsem0, dsem1):
    c = lax.axis_index("c")
    s = lax.axis_index("s")
    base = (c * NSUB + s) * EPT
    dsems = (dsem0, dsem1)
    for k in range(40):
        zbuf_v[pl.ds(k * 16, 16)] = jnp.zeros((16,), jnp.float32)
    for k in range(CHUNK // 16):
        ones_v[pl.ds(k * 16, 16)] = jnp.ones((16,), jnp.float32)

    @pl.when(s < NSUB - 1)
    def _():
        pltpu.sync_copy(zbuf_v, deg_sh.at[pl.ds(s * 640, 640)])

    @pl.when(s == NSUB - 1)
    def _():
        pltpu.sync_copy(zbuf_v.at[pl.ds(0, 512)], deg_sh.at[pl.ds(9600, 512)])

    plsc.subcore_barrier()

    def dfire(j, b):
        pltpu.async_copy(dst_hbm.at[pl.ds(base + j * CHUNK, CHUNK)],
                         didx_b.at[b], dsems[b])

    def dwait(b):
        pltpu.make_async_copy(dst_hbm.at[pl.ds(base, CHUNK)],
                              didx_b.at[b], dsems[b]).wait()

    dfire(0, 0)
    dfire(1, 1)

    def body(i, carry):
        j = 2 * i
        dwait(0)
        pltpu.sync_copy(ones_v, deg_sh.at[didx_b.at[0]], add=True)
        dfire(j + 2, 0)
        dwait(1)
        pltpu.sync_copy(ones_v, deg_sh.at[didx_b.at[1]], add=True)
        dfire(j + 3, 1)
        return carry

    lax.fori_loop(0, NCHUNKS // 2 - 1, body, 0)
    dwait(0)
    pltpu.sync_copy(ones_v, deg_sh.at[didx_b.at[0]], add=True)
    dwait(1)
    pltpu.sync_copy(ones_v, deg_sh.at[didx_b.at[1]], add=True)
    plsc.subcore_barrier()

    @pl.when(s < NSUB - 1)
    def _():
        pltpu.sync_copy(deg_sh.at[pl.ds(s * 640, 640)],
                        out_hbm.at[pl.ds(c * NPAD + s * 640, 640)])

    @pl.when(s == NSUB - 1)
    def _():
        pltpu.sync_copy(deg_sh.at[pl.ds(9600, 512)],
                        out_hbm.at[pl.ds(c * NPAD + 9600, 512)])


def _scat_body(p_hbm, src_hbm, dst_hbm, out_hbm,
               sidx_all, didx_b, rows_v, s_sh,
               gsem0, gsem1, gsem2, dsem0, dsem1, dsem2):
    c = lax.axis_index("c")
    s = lax.axis_index("s")
    base = (c * NSUB + s) * EPT
    pltpu.sync_copy(src_hbm.at[pl.ds(base, EPT)], sidx_all)
    gsems = (gsem0, gsem1, gsem2)
    dsems = (dsem0, dsem1, dsem2)

    def dfire(j, b):
        pltpu.async_copy(dst_hbm.at[pl.ds(base + j * CHUNK, CHUNK)],
                         didx_b.at[b], dsems[b])

    def dwait(b):
        pltpu.make_async_copy(dst_hbm.at[pl.ds(base, CHUNK)],
                              didx_b.at[b], dsems[b]).wait()

    def gfire(j, b):
        pltpu.async_copy(p_hbm.at[sidx_all.at[pl.ds(j * CHUNK, CHUNK)]],
                         rows_v.at[b], gsems[b])

    def gwait(b):
        pltpu.make_async_copy(p_hbm.at[sidx_all.at[pl.ds(0, CHUNK)]],
                              rows_v.at[b], gsems[b]).wait()

    def scat(b):
        pltpu.sync_copy(rows_v.at[b], s_sh.at[didx_b.at[b]], add=True)

    dfire(1, 1)
    gfire(1, 1)
    dfire(2, 2)
    gfire(2, 2)

    def zb(i, carry):
        for k in range(8):
            rows_v[0, i, pl.ds(k * 16, 16)] = jnp.zeros((16,), jnp.float32)
        return carry

    lax.fori_loop(0, CHUNK, zb, 0)
    for t in range(RPT // CHUNK):
        pltpu.sync_copy(rows_v.at[0],
                        s_sh.at[pl.ds(s * RPT + t * CHUNK, CHUNK)])
    rem = RPT % CHUNK
    pltpu.sync_copy(rows_v.at[0, pl.ds(0, rem)],
                    s_sh.at[pl.ds(s * RPT + RPT - rem, rem)])
    plsc.subcore_barrier()
    dfire(0, 0)
    gfire(0, 0)

    def body(i, carry):
        j = 3 * i
        for b in range(NBUF):
            gwait(b)
            dwait(b)
            scat(b)
            gfire(j + b + 3, b)
            dfire(j + b + 3, b)
        return carry

    lax.fori_loop(0, NCHUNKS // 3 - 1, body, 0)
    for b in range(NBUF):
        gwait(b)
        dwait(b)
        scat(b)
    plsc.subcore_barrier()
    pltpu.sync_copy(s_sh.at[pl.ds(s * RPT, RPT)],
                    out_hbm.at[c, pl.ds(s * RPT, RPT)])


MROWS = NPAD
MGRID = NPAD // MROWS


def _mlp_body(xb, w1b, b1b, w2b, b2b, dpb, p0b, cbb):
    h = jnp.maximum(
        jnp.dot(xb[...], w1b[...], preferred_element_type=jnp.float32)
        + b1b[...], 0.0)
    h0 = jnp.dot(h, w2b[...], preferred_element_type=jnp.float32) + b2b[...]
    dp = dpb[...]
    deg = dp[0:1, :] + dp[1:2, :]
    dis = lax.rsqrt(deg)
    dcol = jnp.transpose(dis)
    p0b[...] = h0 * dcol
    cbb[...] = jnp.broadcast_to(0.9 * dcol * dcol, (MROWS, FEAT))


def _comb_body(sb, p0b, cbb, ob):
    sblk = sb[...]
    ob[...] = cbb[...] * (sblk[0] + sblk[1]) + ALPHA * p0b[...]


def _fin_body(sb, p0b, cbb, ob):
    sblk = sb[...]
    cb = cbb[...]
    plast = cb * (sblk[0] + sblk[1]) + ALPHA * p0b[...]
    h = plast * lax.rsqrt(cb * (1.0 / 0.9))
    m = jnp.max(h, axis=1, keepdims=True)
    sh = h - m
    lse = jnp.log(jnp.sum(jnp.exp(sh), axis=1, keepdims=True))
    ob[...] = sh - lse


def kernel(x, edge_index, W1, b1, W2, b2):
    src = edge_index[0]
    dst = edge_index[1]
    loop = jnp.arange(NPAD, dtype=jnp.int32)
    npe = EPAD - EDGES - NPAD
    ar = jnp.arange(npe, dtype=jnp.int32)
    srcp = jnp.concatenate([src, loop, ar % N])
    dstp = jnp.concatenate([dst, loop, N + ar % (NPAD - N)])
    xp = jnp.pad(x, ((0, NPAD - N), (0, 0)))
    b1r = b1.reshape(1, HID)
    b2r = b2.reshape(1, FEAT)

    mesh = plsc.VectorSubcoreMesh(core_axis_name="c", subcore_axis_name="s")

    deg_call = pl.kernel(
        _deg_body,
        out_type=jax.ShapeDtypeStruct((NCORES * NPAD,), jnp.float32),
        mesh=mesh,
        scratch_types=[
            pltpu.VMEM((2, CHUNK), jnp.int32),
            pltpu.VMEM((CHUNK,), jnp.float32),
            pltpu.VMEM((640,), jnp.float32),
            pltpu.VMEM_SHARED((NPAD,), jnp.float32),
            pltpu.SemaphoreType.DMA,
            pltpu.SemaphoreType.DMA,
        ],
    )
    degp = deg_call(dstp).reshape(NCORES, NPAD)

    mlp = pl.pallas_call(
        _mlp_body,
        grid=(MGRID,),
        in_specs=[
            pl.BlockSpec((MROWS, FEAT), lambda i: (i, 0)),
            pl.BlockSpec((FEAT, HID), lambda i: (0, 0)),
            pl.BlockSpec((1, HID), lambda i: (0, 0)),
            pl.BlockSpec((HID, FEAT), lambda i: (0, 0)),
            pl.BlockSpec((1, FEAT), lambda i: (0, 0)),
            pl.BlockSpec((NCORES, MROWS), lambda i: (0, i)),
        ],
        out_specs=[
            pl.BlockSpec((MROWS, FEAT), lambda i: (i, 0)),
            pl.BlockSpec((MROWS, FEAT), lambda i: (i, 0)),
        ],
        out_shape=[
            jax.ShapeDtypeStruct((NPAD, FEAT), jnp.float32),
            jax.ShapeDtypeStruct((NPAD, FEAT), jnp.float32),
        ],
    )
    p0, cB = mlp(xp, W1, b1r, W2, b2r, degp)

    scat = pl.kernel(
        _scat_body,
        out_type=jax.ShapeDtypeStruct((NCORES, NPAD, FEAT), jnp.float32),
        mesh=mesh,
        scratch_types=[
            pltpu.VMEM((EPT,), jnp.int32),
            pltpu.VMEM((NBUF, CHUNK), jnp.int32),
            pltpu.VMEM((NBUF, CHUNK, FEAT), jnp.float32),
            pltpu.VMEM_SHARED((NPAD, FEAT), jnp.float32),
            pltpu.SemaphoreType.DMA,
            pltpu.SemaphoreType.DMA,
            pltpu.SemaphoreType.DMA,
            pltpu.SemaphoreType.DMA,
            pltpu.SemaphoreType.DMA,
            pltpu.SemaphoreType.DMA,
        ],
    )

    ew_specs = [
        pl.BlockSpec((NCORES, CROWS, FEAT), lambda i: (0, i, 0)),
        pl.BlockSpec((CROWS, FEAT), lambda i: (i, 0)),
        pl.BlockSpec((CROWS, FEAT), lambda i: (i, 0)),
    ]
    ew_out = pl.BlockSpec((CROWS, FEAT), lambda i: (i, 0))
    comb = pl.pallas_call(
        _comb_body, grid=(CGRID,), in_specs=ew_specs, out_specs=ew_out,
        out_shape=jax.ShapeDtypeStruct((NPAD, FEAT), jnp.float32))
    fin = pl.pallas_call(
        _fin_body, grid=(CGRID,), in_specs=ew_specs, out_specs=ew_out,
        out_shape=jax.ShapeDtypeStruct((N, FEAT), jnp.float32))

    p = p0
    for _ in range(KHOPS - 1):
        S = scat(p, srcp, dstp)
        p = comb(S, p0, cB)
    S = scat(p, srcp, dstp)
    return fin(S, p0, cB)
